# Initial kernel scaffold; baseline (speedup 1.0000x reference)
import jax
import jax.numpy as jnp
from jax.experimental import pallas as pl


def kernel(x_clients, x_aggregator, edge_ca, edge_ac, W1_ca_l, b1_ca, W1_ca_r, W1_ac_l, b1_ac, W1_ac_r, W2_ca_l, b2_ca, W2_ca_r, W2_ac_l, b2_ac, W2_ac_r, W_lin, b_lin):
    raise NotImplementedError("write your pallas kernel here")


# speedup vs baseline: 1.0138x; 1.0138x over previous
import functools

import jax
import jax.numpy as jnp
from jax import lax
from jax.experimental import pallas as pl
from jax.experimental.pallas import tpu as pltpu
from jax.experimental.pallas import tpu_sc as plsc

N_C = 50000
N_A = 10000
E = 500000
D = 128

NTILES = 32
BLK_E = 256
TPW = 15872
EPAD = NTILES * TPW
EROWS = EPAD // 128
RPT = TPW // 128
NBLK = TPW // BLK_E
NSTR = BLK_E // 128

ACC_CA = 10240
HIST_CA = 10240
DUMP_CA = 10000
CHUNK_AC = 10240
NCH_AC = 5
ACC_AC = 10752
HIST_AC = 50176
DUMP_AC = 50175

BLKR = 1024


def _sc_segsum(nch, chrows, accrows, do_remap):
    mesh = plsc.VectorSubcoreMesh(core_axis_name="c", subcore_axis_name="s",
                                  num_cores=2, num_subcores=16)
    out_type = jax.ShapeDtypeStruct((2, nch * chrows, D), jnp.float32)
    scratch = [
        pltpu.VMEM((BLK_E,), jnp.int32),
        pltpu.VMEM((NSTR, 128), jnp.int32),
        pltpu.VMEM((NSTR, 128), jnp.int32),
        pltpu.VMEM((BLK_E, D), jnp.float32),
        pltpu.VMEM((32, D), jnp.float32),
        pltpu.VMEM_SHARED((accrows, D), jnp.float32),
        pltpu.SemaphoreType.DMA,
    ]

    def body(x_ref, src_ref, dst_ref, out_ref,
             src_st, dst_st, dstl_st, rows, zbuf, acc, gsem):
        c = lax.axis_index("c")
        s = lax.axis_index("s")
        wid = s * 2 + c
        zv = jnp.zeros((16,), jnp.float32)

        def fill_z(r, _):
            for l in range(8):
                zbuf[r, pl.ds(l * 16, 16)] = zv
            return 0
        lax.fori_loop(0, 32, fill_z, 0)

        zr = accrows // 16
        wr = chrows // 16
        for k in range(nch):
            def zacc(i, _):
                pltpu.sync_copy(zbuf, acc.at[pl.ds(s * zr + i * 32, 32), :])
                return 0
            lax.fori_loop(0, zr // 32, zacc, 0)
            plsc.subcore_barrier()

            def blk(j, _):
                e0 = wid * RPT + j * NSTR
                pltpu.sync_copy(src_ref.at[pl.ds(e0 * 128, BLK_E)], src_st)
                pltpu.sync_copy(dst_ref.at[pl.ds(e0, NSTR), :], dst_st)
                if do_remap:
                    for i in range(NSTR):
                        for l in range(8):
                            dv = dst_st[i, pl.ds(l * 16, 16)]
                            loc = dv - k * chrows
                            inb = (loc >= 0) & (loc < chrows)
                            dstl_st[i, pl.ds(l * 16, 16)] = jnp.where(
                                inb, loc, chrows)
                    sidx = dstl_st
                else:
                    sidx = dst_st
                cps = []
                for i in range(NSTR):
                    cp = pltpu.async_copy(
                        x_ref.at[src_st.at[pl.ds(i * 128, 128)]],
                        rows.at[pl.ds(i * 128, 128), :], gsem)
                    cps.append(cp)
                for cp in cps:
                    cp.wait()
                for i in range(NSTR):
                    pltpu.sync_copy(rows.at[pl.ds(i * 128, 128), :],
                                    acc.at[sidx.at[i]], add=True)
                return 0
            lax.fori_loop(0, NBLK, blk, 0)
            plsc.subcore_barrier()

            def wout(i, _):
                r = s * wr + i * 32
                pltpu.sync_copy(acc.at[pl.ds(r, 32), :],
                                out_ref.at[c, pl.ds(k * chrows + r, 32), :])
                return 0
            lax.fori_loop(0, wr // 32, wout, 0)
            plsc.subcore_barrier()

    return pl.kernel(
        body, out_type=out_type, mesh=mesh, scratch_types=tuple(scratch),
        compiler_params=pltpu.CompilerParams(needs_layout_passes=False))


def _sc_hist():
    mesh = plsc.VectorSubcoreMesh(core_axis_name="c", subcore_axis_name="s",
                                  num_cores=2, num_subcores=16)
    out_type = (jax.ShapeDtypeStruct((NTILES, HIST_CA), jnp.float32),
                jax.ShapeDtypeStruct((NTILES, HIST_AC), jnp.float32))
    scratch = [
        pltpu.VMEM((4, 128), jnp.int32),
        pltpu.VMEM((HIST_CA,), jnp.float32),
        pltpu.VMEM((HIST_AC,), jnp.float32),
    ]

    def body(dca_ref, dac_ref, hca_out, hac_out, dst_st, hca, hac):
        c = lax.axis_index("c")
        s = lax.axis_index("s")
        wid = s * 2 + c
        zv = jnp.zeros((16,), jnp.float32)
        ones = jnp.ones((16,), jnp.float32)

        def fill_hca(i, _):
            hca[pl.ds(i * 16, 16)] = zv
            return 0
        lax.fori_loop(0, HIST_CA // 16, fill_hca, 0)

        def fill_hac(i, _):
            hac[pl.ds(i * 16, 16)] = zv
            return 0
        lax.fori_loop(0, HIST_AC // 16, fill_hac, 0)

        for dref, hist in ((dca_ref, hca), (dac_ref, hac)):
            def blk(j, _, dref=dref, hist=hist):
                e0 = wid * RPT + j * 4
                pltpu.sync_copy(dref.at[pl.ds(e0, 4), :], dst_st)
                for i in range(4):
                    for l in range(8):
                        plsc.addupdate_scatter(
                            hist, [dst_st[i, pl.ds(l * 16, 16)]], ones)
                return 0
            lax.fori_loop(0, RPT // 4, blk, 0)
        pltpu.sync_copy(hca, hca_out.at[wid])
        pltpu.sync_copy(hac, hac_out.at[wid])

    return pl.kernel(
        body, out_type=out_type, mesh=mesh, scratch_types=tuple(scratch),
        compiler_params=pltpu.CompilerParams(needs_layout_passes=False))


def _leaky(x):
    return jnp.where(x >= 0, x, 0.1 * x)


def _tc_layer_body(p_ref, h_ref, x_ref, wl_ref, b_ref, wr_ref, o_ref):
    sums = p_ref[0] + p_ref[1]
    cnt = jnp.maximum(jnp.sum(h_ref[...], axis=0), 1.0)
    mean = sums / cnt[:, None]
    z = (jnp.dot(mean, wl_ref[...], preferred_element_type=jnp.float32)
         + b_ref[...][None, :]
         + jnp.dot(x_ref[...], wr_ref[...], preferred_element_type=jnp.float32))
    o_ref[...] = _leaky(z)


def _tc_final_body(p_ref, h_ref, x_ref, wl_ref, b_ref, wr_ref, wlin_ref,
                   blin_ref, o_ref):
    sums = p_ref[0] + p_ref[1]
    cnt = jnp.maximum(jnp.sum(h_ref[...], axis=0), 1.0)
    mean = sums / cnt[:, None]
    z = (jnp.dot(mean, wl_ref[...], preferred_element_type=jnp.float32)
         + b_ref[...][None, :]
         + jnp.dot(x_ref[...], wr_ref[...], preferred_element_type=jnp.float32))
    o_ref[...] = (jnp.dot(_leaky(z), wlin_ref[...],
                          preferred_element_type=jnp.float32)
                  + blin_ref[...][None, :])


def _tc_layer(n_out):
    grid = ((n_out + BLKR - 1) // BLKR,)
    return pl.pallas_call(
        _tc_layer_body,
        grid=grid,
        in_specs=[
            pl.BlockSpec((2, BLKR, D), lambda i: (0, i, 0)),
            pl.BlockSpec((NTILES, BLKR), lambda i: (0, i)),
            pl.BlockSpec((BLKR, D), lambda i: (i, 0)),
            pl.BlockSpec((D, D), lambda i: (0, 0)),
            pl.BlockSpec((D,), lambda i: (0,)),
            pl.BlockSpec((D, D), lambda i: (0, 0)),
        ],
        out_specs=pl.BlockSpec((BLKR, D), lambda i: (i, 0)),
        out_shape=jax.ShapeDtypeStruct((n_out, D), jnp.float32),
    )


def _tc_final(n_out):
    grid = ((n_out + BLKR - 1) // BLKR,)
    return pl.pallas_call(
        _tc_final_body,
        grid=grid,
        in_specs=[
            pl.BlockSpec((2, BLKR, D), lambda i: (0, i, 0)),
            pl.BlockSpec((NTILES, BLKR), lambda i: (0, i)),
            pl.BlockSpec((BLKR, D), lambda i: (i, 0)),
            pl.BlockSpec((D, D), lambda i: (0, 0)),
            pl.BlockSpec((D,), lambda i: (0,)),
            pl.BlockSpec((D, D), lambda i: (0, 0)),
            pl.BlockSpec((D, 1), lambda i: (0, 0)),
            pl.BlockSpec((1,), lambda i: (0,)),
        ],
        out_specs=pl.BlockSpec((BLKR, 1), lambda i: (i, 0)),
        out_shape=jax.ShapeDtypeStruct((n_out, 1), jnp.float32),
    )


_sc_h = _sc_hist()
_sc_ca = _sc_segsum(1, ACC_CA, ACC_CA, False)
_sc_ac = _sc_segsum(NCH_AC, CHUNK_AC, ACC_AC, True)
_t1 = _tc_layer(N_A)
_t2 = _tc_layer(N_C)
_t3 = _tc_final(N_C)


def kernel(x_clients, x_aggregator, edge_ca, edge_ac,
           W1_ca_l, b1_ca, W1_ca_r, W1_ac_l, b1_ac, W1_ac_r,
           W2_ca_l, b2_ca, W2_ca_r, W2_ac_l, b2_ac, W2_ac_r,
           W_lin, b_lin):
    pad = EPAD - E
    ec = edge_ca.astype(jnp.int32)
    ea = edge_ac.astype(jnp.int32)
    src_ca = jnp.pad(ec[0], (0, pad))
    dst_ca = jnp.pad(ec[1], (0, pad),
                     constant_values=DUMP_CA).reshape(EROWS, 128)
    src_ac = jnp.pad(ea[0], (0, pad))
    dst_ac = jnp.pad(ea[1], (0, pad),
                     constant_values=DUMP_AC).reshape(EROWS, 128)

    hist_ca, hist_ac = _sc_h(dst_ca, dst_ac)
    sums_ca = _sc_ca(x_clients, src_ca, dst_ca)
    sums_ac1 = _sc_ac(x_aggregator, src_ac, dst_ac)
    a = _t1(sums_ca, hist_ca, x_aggregator, W1_ca_l, b1_ca, W1_ca_r)
    c = _t2(sums_ac1, hist_ac, x_clients, W1_ac_l, b1_ac, W1_ac_r)
    sums_ac2 = _sc_ac(a, src_ac, dst_ac)
    out = _t3(sums_ac2, hist_ac, c, W2_ac_l, b2_ac, W2_ac_r, W_lin, b_lin)
    return jnp.squeeze(out, axis=1)


# speedup vs baseline: 1.0573x; 1.0430x over previous
import functools

import jax
import jax.numpy as jnp
from jax import lax
from jax.experimental import pallas as pl
from jax.experimental.pallas import tpu as pltpu
from jax.experimental.pallas import tpu_sc as plsc

N_C = 50000
N_A = 10000
E = 500000
D = 128

NTILES = 32
BLK_E = 128
TPW = 15872
EPAD = NTILES * TPW
EROWS = EPAD // 128
RPT = TPW // 128
NBLK = TPW // BLK_E

ACC_CA = 10240
HIST_CA = 10240
DUMP_CA = 10000
CHUNK_AC = 10240
NCH_AC = 5
ACC_AC = 10752
HIST_AC = 50176
DUMP_AC = 50175

BLKR = 1024


def _sc_segsum(nch, chrows, accrows, do_remap):
    mesh = plsc.VectorSubcoreMesh(core_axis_name="c", subcore_axis_name="s",
                                  num_cores=2, num_subcores=16)
    out_type = jax.ShapeDtypeStruct((2, nch * chrows, D), jnp.float32)
    scratch = [
        pltpu.VMEM((BLK_E,), jnp.int32),
        pltpu.VMEM((BLK_E,), jnp.int32),
        pltpu.VMEM((1, 128), jnp.int32),
        pltpu.VMEM((1, 128), jnp.int32),
        pltpu.VMEM((1, 128), jnp.int32),
        pltpu.VMEM((1, 128), jnp.int32),
        pltpu.VMEM((BLK_E, D), jnp.float32),
        pltpu.VMEM((BLK_E, D), jnp.float32),
        pltpu.VMEM((16, D), jnp.float32),
        pltpu.VMEM_SHARED((accrows, D), jnp.float32),
        pltpu.SemaphoreType.DMA,
        pltpu.SemaphoreType.DMA,
        pltpu.SemaphoreType.DMA,
        pltpu.SemaphoreType.DMA,
        pltpu.SemaphoreType.DMA,
        pltpu.SemaphoreType.DMA,
    ]

    def body(x_ref, src_ref, dst_ref, out_ref,
             src0, src1, dst0, dst1, dl0, dl1, rows0, rows1, zbuf, acc,
             is0, is1, gs0, gs1, ss0, ss1):
        src_st = (src0, src1)
        dst_st = (dst0, dst1)
        dstl_st = (dl0, dl1)
        rows = (rows0, rows1)
        isem = (is0, is1)
        gsem = (gs0, gs1)
        ssem = (ss0, ss1)
        c = lax.axis_index("c")
        s = lax.axis_index("s")
        wid = s * 2 + c
        base = wid * RPT
        zv = jnp.zeros((16,), jnp.float32)

        def fill_z(r, _):
            for l in range(8):
                zbuf[r, pl.ds(l * 16, 16)] = zv
            return 0
        lax.fori_loop(0, 16, fill_z, 0)

        def idx_cps(blk, b):
            return (
                pltpu.make_async_copy(
                    src_ref.at[pl.ds((base + blk) * 128, BLK_E)],
                    src_st[b], isem[b]),
                pltpu.make_async_copy(
                    dst_ref.at[pl.ds(base + blk, 1), :], dst_st[b], isem[b]),
            )

        def start_idx(blk, b):
            for cp in idx_cps(blk, b):
                cp.start()

        def sidx_ref(b):
            return dstl_st[b].at[0] if do_remap else dst_st[b].at[0]

        def scat_cp(b):
            return pltpu.make_async_copy(rows[b], acc.at[sidx_ref(b)],
                                         ssem[b])

        def do_block(blk, b, k, first):
            for cp in idx_cps(blk, b):
                cp.wait()
            if not first:
                scat_cp(b).wait()
            if do_remap:
                for l in range(8):
                    dv = dst_st[b][0, pl.ds(l * 16, 16)]
                    loc = dv - k * chrows
                    inb = (loc >= 0) & (loc < chrows)
                    dstl_st[b][0, pl.ds(l * 16, 16)] = jnp.where(
                        inb, loc, chrows)
            g = pltpu.make_async_copy(x_ref.at[src_st[b]], rows[b], gsem[b])
            g.start()
            g.wait()
            if first:
                start_idx(blk + 2, b)
            else:
                @pl.when(blk + 2 < NBLK)
                def _():
                    start_idx(blk + 2, b)
            pltpu.async_copy(rows[b], acc.at[sidx_ref(b)], ssem[b], add=True)

        zr = accrows // 16
        wr = chrows // 16
        for k in range(nch):
            def zacc(i, _):
                pltpu.sync_copy(zbuf, acc.at[pl.ds(s * zr + i * 16, 16), :])
                return 0
            lax.fori_loop(0, zr // 16, zacc, 0)
            plsc.subcore_barrier()

            start_idx(0, 0)
            start_idx(1, 1)
            do_block(0, 0, k, True)
            do_block(1, 1, k, True)

            def pair(m, _):
                do_block(2 * m, 0, k, False)
                do_block(2 * m + 1, 1, k, False)
                return 0
            lax.fori_loop(1, NBLK // 2, pair, 0)
            scat_cp(0).wait()
            scat_cp(1).wait()
            plsc.subcore_barrier()

            def wout(i, _):
                r = s * wr + i * 16
                pltpu.sync_copy(acc.at[pl.ds(r, 16), :],
                                out_ref.at[c, pl.ds(k * chrows + r, 16), :])
                return 0
            lax.fori_loop(0, wr // 16, wout, 0)
            plsc.subcore_barrier()

    return pl.kernel(
        body, out_type=out_type, mesh=mesh, scratch_types=tuple(scratch),
        compiler_params=pltpu.CompilerParams(needs_layout_passes=False))


def _sc_hist():
    mesh = plsc.VectorSubcoreMesh(core_axis_name="c", subcore_axis_name="s",
                                  num_cores=2, num_subcores=16)
    out_type = (jax.ShapeDtypeStruct((NTILES, HIST_CA), jnp.float32),
                jax.ShapeDtypeStruct((NTILES, HIST_AC), jnp.float32))
    scratch = [
        pltpu.VMEM((4, 128), jnp.int32),
        pltpu.VMEM((HIST_CA,), jnp.float32),
        pltpu.VMEM((HIST_AC,), jnp.float32),
    ]

    def body(dca_ref, dac_ref, hca_out, hac_out, dst_st, hca, hac):
        c = lax.axis_index("c")
        s = lax.axis_index("s")
        wid = s * 2 + c
        zv = jnp.zeros((16,), jnp.float32)
        ones = jnp.ones((16,), jnp.float32)

        def fill_hca(i, _):
            hca[pl.ds(i * 16, 16)] = zv
            return 0
        lax.fori_loop(0, HIST_CA // 16, fill_hca, 0)

        def fill_hac(i, _):
            hac[pl.ds(i * 16, 16)] = zv
            return 0
        lax.fori_loop(0, HIST_AC // 16, fill_hac, 0)

        for dref, hist in ((dca_ref, hca), (dac_ref, hac)):
            def blk(j, _, dref=dref, hist=hist):
                e0 = wid * RPT + j * 4
                pltpu.sync_copy(dref.at[pl.ds(e0, 4), :], dst_st)
                for i in range(4):
                    for l in range(8):
                        plsc.addupdate_scatter(
                            hist, [dst_st[i, pl.ds(l * 16, 16)]], ones)
                return 0
            lax.fori_loop(0, RPT // 4, blk, 0)
        pltpu.sync_copy(hca, hca_out.at[wid])
        pltpu.sync_copy(hac, hac_out.at[wid])

    return pl.kernel(
        body, out_type=out_type, mesh=mesh, scratch_types=tuple(scratch),
        compiler_params=pltpu.CompilerParams(needs_layout_passes=False))


def _leaky(x):
    return jnp.where(x >= 0, x, 0.1 * x)


def _tc_layer_body(p_ref, h_ref, x_ref, wl_ref, b_ref, wr_ref, o_ref):
    sums = p_ref[0] + p_ref[1]
    cnt = jnp.maximum(jnp.sum(h_ref[...], axis=0), 1.0)
    mean = sums / cnt[:, None]
    z = (jnp.dot(mean, wl_ref[...], preferred_element_type=jnp.float32)
         + b_ref[...][None, :]
         + jnp.dot(x_ref[...], wr_ref[...], preferred_element_type=jnp.float32))
    o_ref[...] = _leaky(z)


def _tc_final_body(p_ref, h_ref, x_ref, wl_ref, b_ref, wr_ref, wlin_ref,
                   blin_ref, o_ref):
    sums = p_ref[0] + p_ref[1]
    cnt = jnp.maximum(jnp.sum(h_ref[...], axis=0), 1.0)
    mean = sums / cnt[:, None]
    z = (jnp.dot(mean, wl_ref[...], preferred_element_type=jnp.float32)
         + b_ref[...][None, :]
         + jnp.dot(x_ref[...], wr_ref[...], preferred_element_type=jnp.float32))
    o_ref[...] = (jnp.dot(_leaky(z), wlin_ref[...],
                          preferred_element_type=jnp.float32)
                  + blin_ref[...][None, :])


def _tc_layer(n_out):
    grid = ((n_out + BLKR - 1) // BLKR,)
    return pl.pallas_call(
        _tc_layer_body,
        grid=grid,
        in_specs=[
            pl.BlockSpec((2, BLKR, D), lambda i: (0, i, 0)),
            pl.BlockSpec((NTILES, BLKR), lambda i: (0, i)),
            pl.BlockSpec((BLKR, D), lambda i: (i, 0)),
            pl.BlockSpec((D, D), lambda i: (0, 0)),
            pl.BlockSpec((D,), lambda i: (0,)),
            pl.BlockSpec((D, D), lambda i: (0, 0)),
        ],
        out_specs=pl.BlockSpec((BLKR, D), lambda i: (i, 0)),
        out_shape=jax.ShapeDtypeStruct((n_out, D), jnp.float32),
    )


def _tc_final(n_out):
    grid = ((n_out + BLKR - 1) // BLKR,)
    return pl.pallas_call(
        _tc_final_body,
        grid=grid,
        in_specs=[
            pl.BlockSpec((2, BLKR, D), lambda i: (0, i, 0)),
            pl.BlockSpec((NTILES, BLKR), lambda i: (0, i)),
            pl.BlockSpec((BLKR, D), lambda i: (i, 0)),
            pl.BlockSpec((D, D), lambda i: (0, 0)),
            pl.BlockSpec((D,), lambda i: (0,)),
            pl.BlockSpec((D, D), lambda i: (0, 0)),
            pl.BlockSpec((D, 1), lambda i: (0, 0)),
            pl.BlockSpec((1,), lambda i: (0,)),
        ],
        out_specs=pl.BlockSpec((BLKR, 1), lambda i: (i, 0)),
        out_shape=jax.ShapeDtypeStruct((n_out, 1), jnp.float32),
    )


_sc_h = _sc_hist()
_sc_ca = _sc_segsum(1, ACC_CA, ACC_CA, False)
_sc_ac = _sc_segsum(NCH_AC, CHUNK_AC, ACC_AC, True)
_t1 = _tc_layer(N_A)
_t2 = _tc_layer(N_C)
_t3 = _tc_final(N_C)


def kernel(x_clients, x_aggregator, edge_ca, edge_ac,
           W1_ca_l, b1_ca, W1_ca_r, W1_ac_l, b1_ac, W1_ac_r,
           W2_ca_l, b2_ca, W2_ca_r, W2_ac_l, b2_ac, W2_ac_r,
           W_lin, b_lin):
    pad = EPAD - E
    ec = edge_ca.astype(jnp.int32)
    ea = edge_ac.astype(jnp.int32)
    src_ca = jnp.pad(ec[0], (0, pad))
    dst_ca = jnp.pad(ec[1], (0, pad),
                     constant_values=DUMP_CA).reshape(EROWS, 128)
    src_ac = jnp.pad(ea[0], (0, pad))
    dst_ac = jnp.pad(ea[1], (0, pad),
                     constant_values=DUMP_AC).reshape(EROWS, 128)

    hist_ca, hist_ac = _sc_h(dst_ca, dst_ac)
    sums_ca = _sc_ca(x_clients, src_ca, dst_ca)
    sums_ac1 = _sc_ac(x_aggregator, src_ac, dst_ac)
    a = _t1(sums_ca, hist_ca, x_aggregator, W1_ca_l, b1_ca, W1_ca_r)
    c = _t2(sums_ac1, hist_ac, x_clients, W1_ac_l, b1_ac, W1_ac_r)
    sums_ac2 = _sc_ac(a, src_ac, dst_ac)
    out = _t3(sums_ac2, hist_ac, c, W2_ac_l, b2_ac, W2_ac_r, W_lin, b_lin)
    return jnp.squeeze(out, axis=1)


# speedup vs baseline: 2.5720x; 2.4326x over previous
import functools

import jax
import jax.numpy as jnp
from jax import lax
from jax.experimental import pallas as pl
from jax.experimental.pallas import tpu as pltpu
from jax.experimental.pallas import tpu_sc as plsc

N_C = 50000
N_A = 10000
E = 500000
D = 128

NTILES = 32
BLK_E = 128
TPW = 15872
EPAD = NTILES * TPW
EROWS = EPAD // 128
RPT = TPW // 128
NBLK = TPW // BLK_E

ACC_CA = 10240
HIST_CA = 10240
DUMP_CA = 10000
CHUNK_AC = 10240
NCH_AC = 5
ACC_AC = 10752
HIST_AC = 50176
DUMP_AC = 50175

BLKR = 1024


def _sc_segsum(nch, chrows, accrows, do_remap):
    mesh = plsc.VectorSubcoreMesh(core_axis_name="c", subcore_axis_name="s",
                                  num_cores=2, num_subcores=16)
    out_type = jax.ShapeDtypeStruct((2, nch * chrows, D), jnp.float32)
    scratch = [
        pltpu.VMEM((BLK_E,), jnp.int32),
        pltpu.VMEM((BLK_E,), jnp.int32),
        pltpu.VMEM((1, 128), jnp.int32),
        pltpu.VMEM((1, 128), jnp.int32),
        pltpu.VMEM((1, 128), jnp.int32),
        pltpu.VMEM((1, 128), jnp.int32),
        pltpu.VMEM((BLK_E, D), jnp.float32),
        pltpu.VMEM((BLK_E, D), jnp.float32),
        pltpu.VMEM((16, D), jnp.float32),
        pltpu.VMEM_SHARED((accrows, D), jnp.float32),
        pltpu.SemaphoreType.DMA,
        pltpu.SemaphoreType.DMA,
        pltpu.SemaphoreType.DMA,
        pltpu.SemaphoreType.DMA,
        pltpu.SemaphoreType.DMA,
        pltpu.SemaphoreType.DMA,
    ]

    def body(x_ref, src_ref, dst_ref, out_ref,
             src0, src1, dst0, dst1, dl0, dl1, rows0, rows1, zbuf, acc,
             is0, is1, gs0, gs1, ss0, ss1):
        src_st = (src0, src1)
        dst_st = (dst0, dst1)
        dstl_st = (dl0, dl1)
        rows = (rows0, rows1)
        isem = (is0, is1)
        gsem = (gs0, gs1)
        ssem = (ss0, ss1)
        c = lax.axis_index("c")
        s = lax.axis_index("s")
        wid = s * 2 + c
        base = wid * RPT
        zv = jnp.zeros((16,), jnp.float32)

        def fill_z(r, _):
            for l in range(8):
                zbuf[r, pl.ds(l * 16, 16)] = zv
            return 0
        lax.fori_loop(0, 16, fill_z, 0)

        def idx_cps(blk, b):
            return (
                pltpu.make_async_copy(
                    src_ref.at[pl.ds((base + blk) * 128, BLK_E)],
                    src_st[b], isem[b]),
                pltpu.make_async_copy(
                    dst_ref.at[pl.ds(base + blk, 1), :], dst_st[b], isem[b]),
            )

        def start_idx(blk, b):
            for cp in idx_cps(blk, b):
                cp.start()

        def sidx_ref(b):
            return dstl_st[b].at[0] if do_remap else dst_st[b].at[0]

        def scat_cp(b):
            return pltpu.make_async_copy(rows[b], acc.at[sidx_ref(b)],
                                         ssem[b])

        def do_block(blk, b, k, first):
            for cp in idx_cps(blk, b):
                cp.wait()
            if not first:
                scat_cp(b).wait()
            if do_remap:
                for l in range(8):
                    dv = dst_st[b][0, pl.ds(l * 16, 16)]
                    loc = dv - k * chrows
                    inb = (loc >= 0) & (loc < chrows)
                    dstl_st[b][0, pl.ds(l * 16, 16)] = jnp.where(
                        inb, loc, chrows)
            g = pltpu.make_async_copy(x_ref.at[src_st[b]], rows[b], gsem[b])
            g.start()
            g.wait()
            if first:
                start_idx(blk + 2, b)
            else:
                @pl.when(blk + 2 < NBLK)
                def _():
                    start_idx(blk + 2, b)
            pltpu.async_copy(rows[b], acc.at[sidx_ref(b)], ssem[b], add=True)

        zr = accrows // 16
        wr = chrows // 16
        for k in range(nch):
            def zacc(i, _):
                pltpu.sync_copy(zbuf, acc.at[pl.ds(s * zr + i * 16, 16), :])
                return 0
            lax.fori_loop(0, zr // 16, zacc, 0)
            plsc.subcore_barrier()

            start_idx(0, 0)
            start_idx(1, 1)
            do_block(0, 0, k, True)
            do_block(1, 1, k, True)

            def pair(m, _):
                do_block(2 * m, 0, k, False)
                do_block(2 * m + 1, 1, k, False)
                return 0
            lax.fori_loop(1, NBLK // 2, pair, 0)
            scat_cp(0).wait()
            scat_cp(1).wait()
            plsc.subcore_barrier()

            def wout(i, _):
                r = s * wr + i * 16
                pltpu.sync_copy(acc.at[pl.ds(r, 16), :],
                                out_ref.at[c, pl.ds(k * chrows + r, 16), :])
                return 0
            lax.fori_loop(0, wr // 16, wout, 0)
            plsc.subcore_barrier()

    return pl.kernel(
        body, out_type=out_type, mesh=mesh, scratch_types=tuple(scratch),
        compiler_params=pltpu.CompilerParams(needs_layout_passes=False))


BINCAP = 16128
PACK_SHIFT = 14


def _sc_bin():
    mesh = plsc.VectorSubcoreMesh(core_axis_name="c", subcore_axis_name="s",
                                  num_cores=2, num_subcores=16)
    out_type = (jax.ShapeDtypeStruct((NTILES * NCH_AC, BINCAP), jnp.int32),
                jax.ShapeDtypeStruct((NTILES, 128), jnp.int32))
    scratch = (
        [pltpu.VMEM((512,), jnp.int32), pltpu.VMEM((512,), jnp.int32)]
        + [pltpu.VMEM((BINCAP,), jnp.int32) for _ in range(NCH_AC)]
        + [pltpu.VMEM((16,), jnp.int32)]
    )

    def body(src_ref, dst_ref, binned_out, counts_out, src_st, dst_st,
             *bufs_and_cnt):
        bufs = bufs_and_cnt[:NCH_AC]
        cnt_st = bufs_and_cnt[NCH_AC]
        c = lax.axis_index("c")
        s = lax.axis_index("s")
        wid = s * 2 + c
        base = wid * TPW
        iota = lax.iota(jnp.int32, 16)

        def stage(t, pos):
            pltpu.sync_copy(src_ref.at[pl.ds(base + t * 512, 512)], src_st)
            pltpu.sync_copy(dst_ref.at[pl.ds(base + t * 512, 512)], dst_st)
            pos = list(pos)
            for v in range(32):
                sv = src_st[pl.ds(v * 16, 16)]
                dv = dst_st[pl.ds(v * 16, 16)]
                ch = jnp.zeros((16,), jnp.int32)
                for k in range(1, NCH_AC):
                    ch = ch + (dv >= k * CHUNK_AC).astype(jnp.int32)
                loc = dv - ch * CHUNK_AC
                packed = lax.shift_left(sv, PACK_SHIFT) | loc
                for k in range(NCH_AC):
                    mask = ch == k
                    plsc.store_compressed(bufs[k].at[pl.ds(pos[k], 16)],
                                          packed, mask=mask)
                    pos[k] = pos[k] + jnp.sum(mask.astype(jnp.int32))
            return tuple(pos)

        pos = lax.fori_loop(0, TPW // 512, stage,
                            tuple(jnp.int32(0) for _ in range(NCH_AC)))

        dummy = jnp.full((16,), CHUNK_AC, jnp.int32)
        cvec = jnp.zeros((16,), jnp.int32)
        for k in range(NCH_AC):
            for t in range(9):
                bufs[k][pl.ds(pos[k] + t * 16, 16)] = dummy
            nb = lax.shift_right_logical(pos[k] + 127, 7)

            def flush(j, _, k=k):
                pltpu.sync_copy(
                    bufs[k].at[pl.ds(j * 128, 128)],
                    binned_out.at[wid * NCH_AC + k, pl.ds(j * 128, 128)])
                return 0
            lax.fori_loop(0, nb, flush, 0)
            cvec = jnp.where(iota == k, pos[k], cvec)
        cnt_st[pl.ds(0, 16)] = cvec
        pltpu.sync_copy(cnt_st.at[pl.ds(0, 8)], counts_out.at[wid, pl.ds(0, 8)])

    return pl.kernel(
        body, out_type=out_type, mesh=mesh, scratch_types=tuple(scratch),
        compiler_params=pltpu.CompilerParams(needs_layout_passes=False))


def _sc_segsum_binned():
    mesh = plsc.VectorSubcoreMesh(core_axis_name="c", subcore_axis_name="s",
                                  num_cores=2, num_subcores=16)
    out_type = jax.ShapeDtypeStruct((2, NCH_AC * CHUNK_AC, D), jnp.float32)
    scratch = [
        pltpu.VMEM((128,), jnp.int32),
        pltpu.VMEM((128,), jnp.int32),
        pltpu.VMEM((128,), jnp.int32),
        pltpu.VMEM((128,), jnp.int32),
        pltpu.VMEM((1, 128), jnp.int32),
        pltpu.VMEM((1, 128), jnp.int32),
        pltpu.VMEM((128, D), jnp.float32),
        pltpu.VMEM((128, D), jnp.float32),
        pltpu.VMEM((16, D), jnp.float32),
        pltpu.VMEM((16,), jnp.int32),
        pltpu.VMEM_SHARED((ACC_AC, D), jnp.float32),
        pltpu.SemaphoreType.DMA,
        pltpu.SemaphoreType.DMA,
        pltpu.SemaphoreType.DMA,
        pltpu.SemaphoreType.DMA,
        pltpu.SemaphoreType.DMA,
        pltpu.SemaphoreType.DMA,
    ]

    def body(x_ref, binned_ref, counts_ref, out_ref,
             pk0, pk1, sr0, sr1, dl0, dl1, rows0, rows1, zbuf, cnt_st, acc,
             is0, is1, gs0, gs1, ss0, ss1):
        pk_st = (pk0, pk1)
        src_st = (sr0, sr1)
        dstl_st = (dl0, dl1)
        rows = (rows0, rows1)
        isem = (is0, is1)
        gsem = (gs0, gs1)
        ssem = (ss0, ss1)
        c = lax.axis_index("c")
        s = lax.axis_index("s")
        w = s * 2 + c
        iota = lax.iota(jnp.int32, 16)
        zv = jnp.zeros((16,), jnp.float32)

        def fill_z(r, _):
            for l in range(8):
                zbuf[r, pl.ds(l * 16, 16)] = zv
            return 0
        lax.fori_loop(0, 16, fill_z, 0)

        zr = ACC_AC // 16
        wr = CHUNK_AC // 16
        for k in range(NCH_AC):
            def idx_cp(blk, b, k=k):
                return pltpu.make_async_copy(
                    binned_ref.at[w * NCH_AC + k, pl.ds(blk * 128, 128)],
                    pk_st[b], isem[b])

            def scat_cp(b):
                return pltpu.make_async_copy(
                    rows[b], acc.at[dstl_st[b].at[0]], ssem[b])

            def do_block(blk, b, nblk, first):
                idx_cp(blk, b).wait()
                if not first:
                    scat_cp(b).wait()
                for l in range(8):
                    pkv = pk_st[b][pl.ds(l * 16, 16)]
                    src_st[b][pl.ds(l * 16, 16)] = lax.shift_right_logical(
                        pkv, PACK_SHIFT)
                    dstl_st[b][0, pl.ds(l * 16, 16)] = pkv & (
                        (1 << PACK_SHIFT) - 1)
                g = pltpu.make_async_copy(x_ref.at[src_st[b]], rows[b],
                                          gsem[b])
                g.start()
                g.wait()

                @pl.when(blk + 2 < nblk)
                def _():
                    idx_cp(blk + 2, b).start()
                pltpu.async_copy(rows[b], acc.at[dstl_st[b].at[0]], ssem[b],
                                 add=True)

            def zacc(i, _):
                pltpu.sync_copy(zbuf, acc.at[pl.ds(s * zr + i * 16, 16), :])
                return 0
            lax.fori_loop(0, zr // 16, zacc, 0)
            plsc.subcore_barrier()

            pltpu.sync_copy(counts_ref.at[w, pl.ds(0, 8)], cnt_st.at[pl.ds(0, 8)])
            cntk = jnp.sum(jnp.where(iota == k, cnt_st[pl.ds(0, 16)], 0))
            nblk = lax.shift_right_logical(cntk + 127, 7)

            @pl.when(nblk >= 1)
            def _():
                idx_cp(0, 0).start()

            @pl.when(nblk >= 2)
            def _():
                idx_cp(1, 1).start()

            @pl.when(nblk >= 1)
            def _():
                do_block(0, 0, nblk, True)

            @pl.when(nblk >= 2)
            def _():
                do_block(1, 1, nblk, True)

            def pair(m, _):
                do_block(2 * m, 0, nblk, False)

                @pl.when(2 * m + 1 < nblk)
                def _():
                    do_block(2 * m + 1, 1, nblk, False)
                return 0
            lax.fori_loop(1, (nblk + 1) // 2, pair, 0)

            @pl.when(nblk >= 1)
            def _():
                scat_cp(0).wait()

            @pl.when(nblk >= 2)
            def _():
                scat_cp(1).wait()
            plsc.subcore_barrier()

            def wout(i, _):
                r = s * wr + i * 16
                pltpu.sync_copy(acc.at[pl.ds(r, 16), :],
                                out_ref.at[c, pl.ds(k * CHUNK_AC + r, 16), :])
                return 0
            lax.fori_loop(0, wr // 16, wout, 0)
            plsc.subcore_barrier()

    return pl.kernel(
        body, out_type=out_type, mesh=mesh, scratch_types=tuple(scratch),
        compiler_params=pltpu.CompilerParams(needs_layout_passes=False))


def _sc_hist():
    mesh = plsc.VectorSubcoreMesh(core_axis_name="c", subcore_axis_name="s",
                                  num_cores=2, num_subcores=16)
    out_type = (jax.ShapeDtypeStruct((NTILES, HIST_CA), jnp.float32),
                jax.ShapeDtypeStruct((NTILES, HIST_AC), jnp.float32))
    scratch = [
        pltpu.VMEM((4, 128), jnp.int32),
        pltpu.VMEM((HIST_CA,), jnp.float32),
        pltpu.VMEM((HIST_AC,), jnp.float32),
    ]

    def body(dca_ref, dac_ref, hca_out, hac_out, dst_st, hca, hac):
        c = lax.axis_index("c")
        s = lax.axis_index("s")
        wid = s * 2 + c
        zv = jnp.zeros((16,), jnp.float32)
        ones = jnp.ones((16,), jnp.float32)

        def fill_hca(i, _):
            hca[pl.ds(i * 16, 16)] = zv
            return 0
        lax.fori_loop(0, HIST_CA // 16, fill_hca, 0)

        def fill_hac(i, _):
            hac[pl.ds(i * 16, 16)] = zv
            return 0
        lax.fori_loop(0, HIST_AC // 16, fill_hac, 0)

        for dref, hist in ((dca_ref, hca), (dac_ref, hac)):
            def blk(j, _, dref=dref, hist=hist):
                e0 = wid * RPT + j * 4
                pltpu.sync_copy(dref.at[pl.ds(e0, 4), :], dst_st)
                for i in range(4):
                    for l in range(8):
                        plsc.addupdate_scatter(
                            hist, [dst_st[i, pl.ds(l * 16, 16)]], ones)
                return 0
            lax.fori_loop(0, RPT // 4, blk, 0)
        pltpu.sync_copy(hca, hca_out.at[wid])
        pltpu.sync_copy(hac, hac_out.at[wid])

    return pl.kernel(
        body, out_type=out_type, mesh=mesh, scratch_types=tuple(scratch),
        compiler_params=pltpu.CompilerParams(needs_layout_passes=False))


def _leaky(x):
    return jnp.where(x >= 0, x, 0.1 * x)


def _tc_layer_body(p_ref, h_ref, x_ref, wl_ref, b_ref, wr_ref, o_ref):
    sums = p_ref[0] + p_ref[1]
    cnt = jnp.maximum(jnp.sum(h_ref[...], axis=0), 1.0)
    mean = sums / cnt[:, None]
    z = (jnp.dot(mean, wl_ref[...], preferred_element_type=jnp.float32)
         + b_ref[...][None, :]
         + jnp.dot(x_ref[...], wr_ref[...], preferred_element_type=jnp.float32))
    o_ref[...] = _leaky(z)


def _tc_final_body(p_ref, h_ref, x_ref, wl_ref, b_ref, wr_ref, wlin_ref,
                   blin_ref, o_ref):
    sums = p_ref[0] + p_ref[1]
    cnt = jnp.maximum(jnp.sum(h_ref[...], axis=0), 1.0)
    mean = sums / cnt[:, None]
    z = (jnp.dot(mean, wl_ref[...], preferred_element_type=jnp.float32)
         + b_ref[...][None, :]
         + jnp.dot(x_ref[...], wr_ref[...], preferred_element_type=jnp.float32))
    o_ref[...] = (jnp.dot(_leaky(z), wlin_ref[...],
                          preferred_element_type=jnp.float32)
                  + blin_ref[...][None, :])


def _tc_layer(n_out):
    grid = ((n_out + BLKR - 1) // BLKR,)
    return pl.pallas_call(
        _tc_layer_body,
        grid=grid,
        in_specs=[
            pl.BlockSpec((2, BLKR, D), lambda i: (0, i, 0)),
            pl.BlockSpec((NTILES, BLKR), lambda i: (0, i)),
            pl.BlockSpec((BLKR, D), lambda i: (i, 0)),
            pl.BlockSpec((D, D), lambda i: (0, 0)),
            pl.BlockSpec((D,), lambda i: (0,)),
            pl.BlockSpec((D, D), lambda i: (0, 0)),
        ],
        out_specs=pl.BlockSpec((BLKR, D), lambda i: (i, 0)),
        out_shape=jax.ShapeDtypeStruct((n_out, D), jnp.float32),
    )


def _tc_final(n_out):
    grid = ((n_out + BLKR - 1) // BLKR,)
    return pl.pallas_call(
        _tc_final_body,
        grid=grid,
        in_specs=[
            pl.BlockSpec((2, BLKR, D), lambda i: (0, i, 0)),
            pl.BlockSpec((NTILES, BLKR), lambda i: (0, i)),
            pl.BlockSpec((BLKR, D), lambda i: (i, 0)),
            pl.BlockSpec((D, D), lambda i: (0, 0)),
            pl.BlockSpec((D,), lambda i: (0,)),
            pl.BlockSpec((D, D), lambda i: (0, 0)),
            pl.BlockSpec((D, 1), lambda i: (0, 0)),
            pl.BlockSpec((1,), lambda i: (0,)),
        ],
        out_specs=pl.BlockSpec((BLKR, 1), lambda i: (i, 0)),
        out_shape=jax.ShapeDtypeStruct((n_out, 1), jnp.float32),
    )


_sc_h = _sc_hist()
_sc_ca = _sc_segsum(1, ACC_CA, ACC_CA, False)
_sc_b = _sc_bin()
_sc_ac = _sc_segsum_binned()
_t1 = _tc_layer(N_A)
_t2 = _tc_layer(N_C)
_t3 = _tc_final(N_C)


def kernel(x_clients, x_aggregator, edge_ca, edge_ac,
           W1_ca_l, b1_ca, W1_ca_r, W1_ac_l, b1_ac, W1_ac_r,
           W2_ca_l, b2_ca, W2_ca_r, W2_ac_l, b2_ac, W2_ac_r,
           W_lin, b_lin):
    pad = EPAD - E
    ec = edge_ca.astype(jnp.int32)
    ea = edge_ac.astype(jnp.int32)
    src_ca = jnp.pad(ec[0], (0, pad))
    dst_ca = jnp.pad(ec[1], (0, pad),
                     constant_values=DUMP_CA).reshape(EROWS, 128)
    src_ac = jnp.pad(ea[0], (0, pad))
    dst_ac1 = jnp.pad(ea[1], (0, pad), constant_values=DUMP_AC)
    dst_ac = dst_ac1.reshape(EROWS, 128)

    hist_ca, hist_ac = _sc_h(dst_ca, dst_ac)
    binned, bcounts = _sc_b(src_ac, dst_ac1)
    sums_ca = _sc_ca(x_clients, src_ca, dst_ca)
    sums_ac1 = _sc_ac(x_aggregator, binned, bcounts)
    a = _t1(sums_ca, hist_ca, x_aggregator, W1_ca_l, b1_ca, W1_ca_r)
    c = _t2(sums_ac1, hist_ac, x_clients, W1_ac_l, b1_ac, W1_ac_r)
    sums_ac2 = _sc_ac(a, binned, bcounts)
    out = _t3(sums_ac2, hist_ac, c, W2_ac_l, b2_ac, W2_ac_r, W_lin, b_lin)
    return jnp.squeeze(out, axis=1)


# speedup vs baseline: 2.6665x; 1.0368x over previous
import functools

import jax
import jax.numpy as jnp
from jax import lax
from jax.experimental import pallas as pl
from jax.experimental.pallas import tpu as pltpu
from jax.experimental.pallas import tpu_sc as plsc

N_C = 50000
N_A = 10000
E = 500000
D = 128

NTILES = 32
BLK_E = 128
TPW = 15872
EPAD = NTILES * TPW
EROWS = EPAD // 128
RPT = TPW // 128
NBLK = TPW // BLK_E

ACC_CA = 10240
HIST_CA = 10240
DUMP_CA = 10000
CHUNK_AC = 10240
NCH_AC = 5
ACC_AC = 10752
HIST_AC = 50176
DUMP_AC = 50175

BLKR = 1024


def _sc_segsum(nch, chrows, accrows, do_remap):
    mesh = plsc.VectorSubcoreMesh(core_axis_name="c", subcore_axis_name="s",
                                  num_cores=2, num_subcores=16)
    out_type = jax.ShapeDtypeStruct((2, nch * chrows, D), jnp.float32)
    scratch = [
        pltpu.VMEM((BLK_E,), jnp.int32),
        pltpu.VMEM((BLK_E,), jnp.int32),
        pltpu.VMEM((1, 128), jnp.int32),
        pltpu.VMEM((1, 128), jnp.int32),
        pltpu.VMEM((1, 128), jnp.int32),
        pltpu.VMEM((1, 128), jnp.int32),
        pltpu.VMEM((BLK_E, D), jnp.float32),
        pltpu.VMEM((BLK_E, D), jnp.float32),
        pltpu.VMEM((16, D), jnp.float32),
        pltpu.VMEM_SHARED((accrows, D), jnp.float32),
        pltpu.SemaphoreType.DMA,
        pltpu.SemaphoreType.DMA,
        pltpu.SemaphoreType.DMA,
        pltpu.SemaphoreType.DMA,
        pltpu.SemaphoreType.DMA,
        pltpu.SemaphoreType.DMA,
    ]

    def body(x_ref, src_ref, dst_ref, out_ref,
             src0, src1, dst0, dst1, dl0, dl1, rows0, rows1, zbuf, acc,
             is0, is1, gs0, gs1, ss0, ss1):
        src_st = (src0, src1)
        dst_st = (dst0, dst1)
        dstl_st = (dl0, dl1)
        rows = (rows0, rows1)
        isem = (is0, is1)
        gsem = (gs0, gs1)
        ssem = (ss0, ss1)
        c = lax.axis_index("c")
        s = lax.axis_index("s")
        wid = s * 2 + c
        base = wid * RPT
        zv = jnp.zeros((16,), jnp.float32)

        def fill_z(r, _):
            for l in range(8):
                zbuf[r, pl.ds(l * 16, 16)] = zv
            return 0
        lax.fori_loop(0, 16, fill_z, 0)

        def idx_cps(blk, b):
            return (
                pltpu.make_async_copy(
                    src_ref.at[pl.ds((base + blk) * 128, BLK_E)],
                    src_st[b], isem[b]),
                pltpu.make_async_copy(
                    dst_ref.at[pl.ds(base + blk, 1), :], dst_st[b], isem[b]),
            )

        def start_idx(blk, b):
            for cp in idx_cps(blk, b):
                cp.start()

        def sidx_ref(b):
            return dstl_st[b].at[0] if do_remap else dst_st[b].at[0]

        def scat_cp(b):
            return pltpu.make_async_copy(rows[b], acc.at[sidx_ref(b)],
                                         ssem[b])

        def do_block(blk, b, k, first):
            for cp in idx_cps(blk, b):
                cp.wait()
            if not first:
                scat_cp(b).wait()
            if do_remap:
                for l in range(8):
                    dv = dst_st[b][0, pl.ds(l * 16, 16)]
                    loc = dv - k * chrows
                    inb = (loc >= 0) & (loc < chrows)
                    dstl_st[b][0, pl.ds(l * 16, 16)] = jnp.where(
                        inb, loc, chrows)
            g = pltpu.make_async_copy(x_ref.at[src_st[b]], rows[b], gsem[b])
            g.start()
            g.wait()
            if first:
                start_idx(blk + 2, b)
            else:
                @pl.when(blk + 2 < NBLK)
                def _():
                    start_idx(blk + 2, b)
            pltpu.async_copy(rows[b], acc.at[sidx_ref(b)], ssem[b], add=True)

        zr = accrows // 16
        wr = chrows // 16
        for k in range(nch):
            def zacc(i, _):
                pltpu.sync_copy(zbuf, acc.at[pl.ds(s * zr + i * 16, 16), :])
                return 0
            lax.fori_loop(0, zr // 16, zacc, 0)
            plsc.subcore_barrier()

            start_idx(0, 0)
            start_idx(1, 1)
            do_block(0, 0, k, True)
            do_block(1, 1, k, True)

            def pair(m, _):
                do_block(2 * m, 0, k, False)
                do_block(2 * m + 1, 1, k, False)
                return 0
            lax.fori_loop(1, NBLK // 2, pair, 0)
            scat_cp(0).wait()
            scat_cp(1).wait()
            plsc.subcore_barrier()

            def wout(i, _):
                r = s * wr + i * 16
                pltpu.sync_copy(acc.at[pl.ds(r, 16), :],
                                out_ref.at[c, pl.ds(k * chrows + r, 16), :])
                return 0
            lax.fori_loop(0, wr // 16, wout, 0)
            plsc.subcore_barrier()

    return pl.kernel(
        body, out_type=out_type, mesh=mesh, scratch_types=tuple(scratch),
        compiler_params=pltpu.CompilerParams(needs_layout_passes=False))


BINCAP = 16128
PACK_SHIFT = 14


def _sc_bin():
    mesh = plsc.VectorSubcoreMesh(core_axis_name="c", subcore_axis_name="s",
                                  num_cores=2, num_subcores=16)
    out_type = (jax.ShapeDtypeStruct((NTILES * NCH_AC, BINCAP), jnp.int32),
                jax.ShapeDtypeStruct((NTILES, 128), jnp.int32))
    scratch = (
        [pltpu.VMEM((512,), jnp.int32), pltpu.VMEM((512,), jnp.int32)]
        + [pltpu.VMEM((BINCAP,), jnp.int32) for _ in range(NCH_AC)]
        + [pltpu.VMEM((16,), jnp.int32)]
    )

    def body(src_ref, dst_ref, binned_out, counts_out, src_st, dst_st,
             *bufs_and_cnt):
        bufs = bufs_and_cnt[:NCH_AC]
        cnt_st = bufs_and_cnt[NCH_AC]
        c = lax.axis_index("c")
        s = lax.axis_index("s")
        wid = s * 2 + c
        base = wid * TPW
        iota = lax.iota(jnp.int32, 16)

        def stage(t, pos):
            pltpu.sync_copy(src_ref.at[pl.ds(base + t * 512, 512)], src_st)
            pltpu.sync_copy(dst_ref.at[pl.ds(base + t * 512, 512)], dst_st)
            pos = list(pos)
            for v in range(32):
                sv = src_st[pl.ds(v * 16, 16)]
                dv = dst_st[pl.ds(v * 16, 16)]
                ch = jnp.zeros((16,), jnp.int32)
                for k in range(1, NCH_AC):
                    ch = ch + (dv >= k * CHUNK_AC).astype(jnp.int32)
                loc = dv - ch * CHUNK_AC
                packed = lax.shift_left(sv, PACK_SHIFT) | loc
                for k in range(NCH_AC):
                    mask = ch == k
                    plsc.store_compressed(bufs[k].at[pl.ds(pos[k], 16)],
                                          packed, mask=mask)
                    pos[k] = pos[k] + jnp.sum(mask.astype(jnp.int32))
            return tuple(pos)

        pos = lax.fori_loop(0, TPW // 512, stage,
                            tuple(jnp.int32(0) for _ in range(NCH_AC)))

        dummy = jnp.full((16,), CHUNK_AC, jnp.int32)
        cvec = jnp.zeros((16,), jnp.int32)
        for k in range(NCH_AC):
            for t in range(9):
                bufs[k][pl.ds(pos[k] + t * 16, 16)] = dummy
            nb = lax.shift_right_logical(pos[k] + 127, 7)

            def flush(j, _, k=k):
                pltpu.sync_copy(
                    bufs[k].at[pl.ds(j * 128, 128)],
                    binned_out.at[wid * NCH_AC + k, pl.ds(j * 128, 128)])
                return 0
            lax.fori_loop(0, nb, flush, 0)
            cvec = jnp.where(iota == k, pos[k], cvec)
        cnt_st[pl.ds(0, 16)] = cvec
        pltpu.sync_copy(cnt_st.at[pl.ds(0, 8)], counts_out.at[wid, pl.ds(0, 8)])

    return pl.kernel(
        body, out_type=out_type, mesh=mesh, scratch_types=tuple(scratch),
        compiler_params=pltpu.CompilerParams(needs_layout_passes=False))


def _sc_segsum_binned():
    mesh = plsc.VectorSubcoreMesh(core_axis_name="c", subcore_axis_name="s",
                                  num_cores=2, num_subcores=16)
    out_type = jax.ShapeDtypeStruct((2, NCH_AC * CHUNK_AC, D), jnp.float32)
    scratch = [
        pltpu.VMEM((128,), jnp.int32),
        pltpu.VMEM((128,), jnp.int32),
        pltpu.VMEM((128,), jnp.int32),
        pltpu.VMEM((128,), jnp.int32),
        pltpu.VMEM((1, 128), jnp.int32),
        pltpu.VMEM((1, 128), jnp.int32),
        pltpu.VMEM((128, D), jnp.float32),
        pltpu.VMEM((128, D), jnp.float32),
        pltpu.VMEM((16, D), jnp.float32),
        pltpu.VMEM((16,), jnp.int32),
        pltpu.VMEM_SHARED((ACC_AC, D), jnp.float32),
        pltpu.SemaphoreType.DMA,
        pltpu.SemaphoreType.DMA,
        pltpu.SemaphoreType.DMA,
        pltpu.SemaphoreType.DMA,
        pltpu.SemaphoreType.DMA,
        pltpu.SemaphoreType.DMA,
    ]

    def body(x_ref, binned_ref, counts_ref, out_ref,
             pk0, pk1, sr0, sr1, dl0, dl1, rows0, rows1, zbuf, cnt_st, acc,
             is0, is1, gs0, gs1, ss0, ss1):
        pk_st = (pk0, pk1)
        src_st = (sr0, sr1)
        dstl_st = (dl0, dl1)
        rows = (rows0, rows1)
        isem = (is0, is1)
        gsem = (gs0, gs1)
        ssem = (ss0, ss1)
        c = lax.axis_index("c")
        s = lax.axis_index("s")
        w = s * 2 + c
        iota = lax.iota(jnp.int32, 16)
        zv = jnp.zeros((16,), jnp.float32)

        def fill_z(r, _):
            for l in range(8):
                zbuf[r, pl.ds(l * 16, 16)] = zv
            return 0
        lax.fori_loop(0, 16, fill_z, 0)

        zr = ACC_AC // 16
        wr = CHUNK_AC // 16
        for k in range(NCH_AC):
            def idx_cp(blk, b, k=k):
                return pltpu.make_async_copy(
                    binned_ref.at[w * NCH_AC + k, pl.ds(blk * 128, 128)],
                    pk_st[b], isem[b])

            def scat_cp(b):
                return pltpu.make_async_copy(
                    rows[b], acc.at[dstl_st[b].at[0]], ssem[b])

            def gath_cp(b):
                return pltpu.make_async_copy(x_ref.at[src_st[b]], rows[b],
                                             gsem[b])

            def stage_a(blk, b, nblk, wait_scat):
                idx_cp(blk, b).wait()
                if wait_scat:
                    scat_cp(b).wait()
                for l in range(8):
                    pkv = pk_st[b][pl.ds(l * 16, 16)]
                    src_st[b][pl.ds(l * 16, 16)] = lax.shift_right_logical(
                        pkv, PACK_SHIFT)
                    dstl_st[b][0, pl.ds(l * 16, 16)] = pkv & (
                        (1 << PACK_SHIFT) - 1)

                @pl.when(blk + 2 < nblk)
                def _():
                    idx_cp(blk + 2, b).start()
                gath_cp(b).start()

            def complete(b):
                gath_cp(b).wait()
                pltpu.async_copy(rows[b], acc.at[dstl_st[b].at[0]], ssem[b],
                                 add=True)

            def zacc(i, _):
                pltpu.sync_copy(zbuf, acc.at[pl.ds(s * zr + i * 16, 16), :])
                return 0
            lax.fori_loop(0, zr // 16, zacc, 0)
            plsc.subcore_barrier()

            pltpu.sync_copy(counts_ref.at[w, pl.ds(0, 8)], cnt_st.at[pl.ds(0, 8)])
            cntk = jnp.sum(jnp.where(iota == k, cnt_st[pl.ds(0, 16)], 0))
            nblk = lax.shift_right_logical(cntk + 127, 7)

            @pl.when(nblk >= 1)
            def _():
                idx_cp(0, 0).start()

            @pl.when(nblk >= 2)
            def _():
                idx_cp(1, 1).start()

            @pl.when(nblk >= 1)
            def _():
                stage_a(0, 0, nblk, False)

            @pl.when(nblk >= 2)
            def _():
                stage_a(1, 1, nblk, False)

            def pair(m, _):
                @pl.when(2 * m < nblk)
                def _():
                    complete(0)
                    stage_a(2 * m, 0, nblk, True)

                @pl.when(2 * m + 1 < nblk)
                def _():
                    complete(1)
                    stage_a(2 * m + 1, 1, nblk, True)
                return 0
            lax.fori_loop(1, (nblk + 1) // 2, pair, 0)

            @pl.when(nblk >= 1)
            def _():
                complete(0)

            @pl.when(nblk >= 2)
            def _():
                complete(1)

            @pl.when(nblk >= 1)
            def _():
                scat_cp(0).wait()

            @pl.when(nblk >= 2)
            def _():
                scat_cp(1).wait()
            plsc.subcore_barrier()

            def wout(i, _):
                r = s * wr + i * 16
                pltpu.sync_copy(acc.at[pl.ds(r, 16), :],
                                out_ref.at[c, pl.ds(k * CHUNK_AC + r, 16), :])
                return 0
            lax.fori_loop(0, wr // 16, wout, 0)
            plsc.subcore_barrier()

    return pl.kernel(
        body, out_type=out_type, mesh=mesh, scratch_types=tuple(scratch),
        compiler_params=pltpu.CompilerParams(needs_layout_passes=False))


def _sc_hist():
    mesh = plsc.VectorSubcoreMesh(core_axis_name="c", subcore_axis_name="s",
                                  num_cores=2, num_subcores=16)
    out_type = (jax.ShapeDtypeStruct((NTILES, HIST_CA), jnp.float32),
                jax.ShapeDtypeStruct((NTILES, HIST_AC), jnp.float32))
    scratch = [
        pltpu.VMEM((4, 128), jnp.int32),
        pltpu.VMEM((HIST_CA,), jnp.float32),
        pltpu.VMEM((HIST_AC,), jnp.float32),
    ]

    def body(dca_ref, dac_ref, hca_out, hac_out, dst_st, hca, hac):
        c = lax.axis_index("c")
        s = lax.axis_index("s")
        wid = s * 2 + c
        zv = jnp.zeros((16,), jnp.float32)
        ones = jnp.ones((16,), jnp.float32)

        def fill_hca(i, _):
            hca[pl.ds(i * 16, 16)] = zv
            return 0
        lax.fori_loop(0, HIST_CA // 16, fill_hca, 0)

        def fill_hac(i, _):
            hac[pl.ds(i * 16, 16)] = zv
            return 0
        lax.fori_loop(0, HIST_AC // 16, fill_hac, 0)

        for dref, hist in ((dca_ref, hca), (dac_ref, hac)):
            def blk(j, _, dref=dref, hist=hist):
                e0 = wid * RPT + j * 4
                pltpu.sync_copy(dref.at[pl.ds(e0, 4), :], dst_st)
                for i in range(4):
                    for l in range(8):
                        plsc.addupdate_scatter(
                            hist, [dst_st[i, pl.ds(l * 16, 16)]], ones)
                return 0
            lax.fori_loop(0, RPT // 4, blk, 0)
        pltpu.sync_copy(hca, hca_out.at[wid])
        pltpu.sync_copy(hac, hac_out.at[wid])

    return pl.kernel(
        body, out_type=out_type, mesh=mesh, scratch_types=tuple(scratch),
        compiler_params=pltpu.CompilerParams(needs_layout_passes=False))


def _leaky(x):
    return jnp.where(x >= 0, x, 0.1 * x)


def _tc_layer_body(p_ref, h_ref, x_ref, wl_ref, b_ref, wr_ref, o_ref):
    sums = p_ref[0] + p_ref[1]
    cnt = jnp.maximum(jnp.sum(h_ref[...], axis=0), 1.0)
    mean = sums / cnt[:, None]
    z = (jnp.dot(mean, wl_ref[...], preferred_element_type=jnp.float32)
         + b_ref[...][None, :]
         + jnp.dot(x_ref[...], wr_ref[...], preferred_element_type=jnp.float32))
    o_ref[...] = _leaky(z)


def _tc_final_body(p_ref, h_ref, x_ref, wl_ref, b_ref, wr_ref, wlin_ref,
                   blin_ref, o_ref):
    sums = p_ref[0] + p_ref[1]
    cnt = jnp.maximum(jnp.sum(h_ref[...], axis=0), 1.0)
    mean = sums / cnt[:, None]
    z = (jnp.dot(mean, wl_ref[...], preferred_element_type=jnp.float32)
         + b_ref[...][None, :]
         + jnp.dot(x_ref[...], wr_ref[...], preferred_element_type=jnp.float32))
    o_ref[...] = (jnp.dot(_leaky(z), wlin_ref[...],
                          preferred_element_type=jnp.float32)
                  + blin_ref[...][None, :])


def _tc_layer(n_out):
    grid = ((n_out + BLKR - 1) // BLKR,)
    return pl.pallas_call(
        _tc_layer_body,
        grid=grid,
        in_specs=[
            pl.BlockSpec((2, BLKR, D), lambda i: (0, i, 0)),
            pl.BlockSpec((NTILES, BLKR), lambda i: (0, i)),
            pl.BlockSpec((BLKR, D), lambda i: (i, 0)),
            pl.BlockSpec((D, D), lambda i: (0, 0)),
            pl.BlockSpec((D,), lambda i: (0,)),
            pl.BlockSpec((D, D), lambda i: (0, 0)),
        ],
        out_specs=pl.BlockSpec((BLKR, D), lambda i: (i, 0)),
        out_shape=jax.ShapeDtypeStruct((n_out, D), jnp.float32),
    )


def _tc_final(n_out):
    grid = ((n_out + BLKR - 1) // BLKR,)
    return pl.pallas_call(
        _tc_final_body,
        grid=grid,
        in_specs=[
            pl.BlockSpec((2, BLKR, D), lambda i: (0, i, 0)),
            pl.BlockSpec((NTILES, BLKR), lambda i: (0, i)),
            pl.BlockSpec((BLKR, D), lambda i: (i, 0)),
            pl.BlockSpec((D, D), lambda i: (0, 0)),
            pl.BlockSpec((D,), lambda i: (0,)),
            pl.BlockSpec((D, D), lambda i: (0, 0)),
            pl.BlockSpec((D, 1), lambda i: (0, 0)),
            pl.BlockSpec((1,), lambda i: (0,)),
        ],
        out_specs=pl.BlockSpec((BLKR, 1), lambda i: (i, 0)),
        out_shape=jax.ShapeDtypeStruct((n_out, 1), jnp.float32),
    )


_sc_h = _sc_hist()
_sc_ca = _sc_segsum(1, ACC_CA, ACC_CA, False)
_sc_b = _sc_bin()
_sc_ac = _sc_segsum_binned()
_t1 = _tc_layer(N_A)
_t2 = _tc_layer(N_C)
_t3 = _tc_final(N_C)


def kernel(x_clients, x_aggregator, edge_ca, edge_ac,
           W1_ca_l, b1_ca, W1_ca_r, W1_ac_l, b1_ac, W1_ac_r,
           W2_ca_l, b2_ca, W2_ca_r, W2_ac_l, b2_ac, W2_ac_r,
           W_lin, b_lin):
    pad = EPAD - E
    ec = edge_ca.astype(jnp.int32)
    ea = edge_ac.astype(jnp.int32)
    src_ca = jnp.pad(ec[0], (0, pad))
    dst_ca = jnp.pad(ec[1], (0, pad),
                     constant_values=DUMP_CA).reshape(EROWS, 128)
    src_ac = jnp.pad(ea[0], (0, pad))
    dst_ac1 = jnp.pad(ea[1], (0, pad), constant_values=DUMP_AC)
    dst_ac = dst_ac1.reshape(EROWS, 128)

    hist_ca, hist_ac = _sc_h(dst_ca, dst_ac)
    binned, bcounts = _sc_b(src_ac, dst_ac1)
    sums_ca = _sc_ca(x_clients, src_ca, dst_ca)
    sums_ac1 = _sc_ac(x_aggregator, binned, bcounts)
    a = _t1(sums_ca, hist_ca, x_aggregator, W1_ca_l, b1_ca, W1_ca_r)
    c = _t2(sums_ac1, hist_ac, x_clients, W1_ac_l, b1_ac, W1_ac_r)
    sums_ac2 = _sc_ac(a, binned, bcounts)
    out = _t3(sums_ac2, hist_ac, c, W2_ac_l, b2_ac, W2_ac_r, W_lin, b_lin)
    return jnp.squeeze(out, axis=1)


# speedup vs baseline: 3.0817x; 1.1557x over previous
import functools

import jax
import jax.numpy as jnp
from jax import lax
from jax.experimental import pallas as pl
from jax.experimental.pallas import tpu as pltpu
from jax.experimental.pallas import tpu_sc as plsc

N_C = 50000
N_A = 10000
E = 500000
D = 128

NTILES = 32
BLK_E = 128
TPW = 15872
EPAD = NTILES * TPW
EROWS = EPAD // 128
RPT = TPW // 128
NBLK = TPW // BLK_E

ACC_CA = 10240
HIST_CA = 10240
DUMP_CA = 10000
CHUNK_AC = 10240
NCH_AC = 5
ACC_AC = 10752
HIST_AC = 50176
DUMP_AC = 50175

BLKR = 1024


def _sc_segsum(nch, chrows, accrows, do_remap):
    mesh = plsc.VectorSubcoreMesh(core_axis_name="c", subcore_axis_name="s",
                                  num_cores=2, num_subcores=16)
    out_type = jax.ShapeDtypeStruct((2, nch * chrows, D), jnp.float32)
    scratch = [
        pltpu.VMEM((BLK_E,), jnp.int32),
        pltpu.VMEM((BLK_E,), jnp.int32),
        pltpu.VMEM((1, 128), jnp.int32),
        pltpu.VMEM((1, 128), jnp.int32),
        pltpu.VMEM((1, 128), jnp.int32),
        pltpu.VMEM((1, 128), jnp.int32),
        pltpu.VMEM((BLK_E, D), jnp.float32),
        pltpu.VMEM((BLK_E, D), jnp.float32),
        pltpu.VMEM((16, D), jnp.float32),
        pltpu.VMEM_SHARED((accrows, D), jnp.float32),
        pltpu.SemaphoreType.DMA,
        pltpu.SemaphoreType.DMA,
        pltpu.SemaphoreType.DMA,
        pltpu.SemaphoreType.DMA,
        pltpu.SemaphoreType.DMA,
        pltpu.SemaphoreType.DMA,
        pltpu.SemaphoreType.DMA,
    ]

    def body(x_ref, src_ref, dst_ref, out_ref,
             src0, src1, dst0, dst1, dl0, dl1, rows0, rows1, zbuf, acc,
             is0, is1, gs0, gs1, ss0, ss1, zs):
        src_st = (src0, src1)
        dst_st = (dst0, dst1)
        dstl_st = (dl0, dl1)
        rows = (rows0, rows1)
        isem = (is0, is1)
        gsem = (gs0, gs1)
        ssem = (ss0, ss1)
        c = lax.axis_index("c")
        s = lax.axis_index("s")
        wid = s * 2 + c
        base = wid * RPT
        zv = jnp.zeros((16,), jnp.float32)

        def fill_z(r, _):
            for l in range(8):
                zbuf[r, pl.ds(l * 16, 16)] = zv
            return 0
        lax.fori_loop(0, 16, fill_z, 0)

        def idx_cps(blk, b):
            return (
                pltpu.make_async_copy(
                    src_ref.at[pl.ds((base + blk) * 128, BLK_E)],
                    src_st[b], isem[b]),
                pltpu.make_async_copy(
                    dst_ref.at[pl.ds(base + blk, 1), :], dst_st[b], isem[b]),
            )

        def start_idx(blk, b):
            for cp in idx_cps(blk, b):
                cp.start()

        def sidx_ref(b):
            return dstl_st[b].at[0] if do_remap else dst_st[b].at[0]

        def scat_cp(b):
            return pltpu.make_async_copy(rows[b], acc.at[sidx_ref(b)],
                                         ssem[b])

        def do_block(blk, b, k, first):
            for cp in idx_cps(blk, b):
                cp.wait()
            if not first:
                scat_cp(b).wait()
            if do_remap:
                for l in range(8):
                    dv = dst_st[b][0, pl.ds(l * 16, 16)]
                    loc = dv - k * chrows
                    inb = (loc >= 0) & (loc < chrows)
                    dstl_st[b][0, pl.ds(l * 16, 16)] = jnp.where(
                        inb, loc, chrows)
            g = pltpu.make_async_copy(x_ref.at[src_st[b]], rows[b], gsem[b])
            g.start()
            g.wait()
            if first:
                start_idx(blk + 2, b)
            else:
                @pl.when(blk + 2 < NBLK)
                def _():
                    start_idx(blk + 2, b)
            pltpu.async_copy(rows[b], acc.at[sidx_ref(b)], ssem[b], add=True)

        wr = chrows // 16
        for k in range(nch):
            def zcp(i):
                return pltpu.make_async_copy(
                    zbuf, acc.at[pl.ds(s * wr + i * 16, 16), :], zs)

            def zstart(i, _):
                zcp(i).start()
                return 0
            lax.fori_loop(0, wr // 16, zstart, 0)

            def zwait(i, _):
                zcp(i).wait()
                return 0
            lax.fori_loop(0, wr // 16, zwait, 0)
            plsc.subcore_barrier()

            start_idx(0, 0)
            start_idx(1, 1)
            do_block(0, 0, k, True)
            do_block(1, 1, k, True)

            def pair(m, _):
                do_block(2 * m, 0, k, False)
                do_block(2 * m + 1, 1, k, False)
                return 0
            lax.fori_loop(1, NBLK // 2, pair, 0)
            scat_cp(0).wait()
            scat_cp(1).wait()
            plsc.subcore_barrier()

            pltpu.sync_copy(
                acc.at[pl.ds(s * wr, wr), :],
                out_ref.at[c, pl.ds(k * chrows + s * wr, wr), :])

    return pl.kernel(
        body, out_type=out_type, mesh=mesh, scratch_types=tuple(scratch),
        compiler_params=pltpu.CompilerParams(needs_layout_passes=False))


BINCAP = 16128
PACK_SHIFT = 14


def _sc_bin():
    mesh = plsc.VectorSubcoreMesh(core_axis_name="c", subcore_axis_name="s",
                                  num_cores=2, num_subcores=16)
    out_type = (jax.ShapeDtypeStruct((NTILES * NCH_AC, BINCAP), jnp.int32),
                jax.ShapeDtypeStruct((NTILES, 128), jnp.int32))
    scratch = (
        [pltpu.VMEM((512,), jnp.int32), pltpu.VMEM((512,), jnp.int32)]
        + [pltpu.VMEM((BINCAP,), jnp.int32) for _ in range(NCH_AC)]
        + [pltpu.VMEM((16,), jnp.int32)]
    )

    def body(src_ref, dst_ref, binned_out, counts_out, src_st, dst_st,
             *bufs_and_cnt):
        bufs = bufs_and_cnt[:NCH_AC]
        cnt_st = bufs_and_cnt[NCH_AC]
        c = lax.axis_index("c")
        s = lax.axis_index("s")
        wid = s * 2 + c
        base = wid * TPW
        iota = lax.iota(jnp.int32, 16)

        def stage(t, pos):
            pltpu.sync_copy(src_ref.at[pl.ds(base + t * 512, 512)], src_st)
            pltpu.sync_copy(dst_ref.at[pl.ds(base + t * 512, 512)], dst_st)
            pos = list(pos)
            for v in range(32):
                sv = src_st[pl.ds(v * 16, 16)]
                dv = dst_st[pl.ds(v * 16, 16)]
                ch = jnp.zeros((16,), jnp.int32)
                for k in range(1, NCH_AC):
                    ch = ch + (dv >= k * CHUNK_AC).astype(jnp.int32)
                loc = dv - ch * CHUNK_AC
                packed = lax.shift_left(sv, PACK_SHIFT) | loc
                for k in range(NCH_AC):
                    mask = ch == k
                    plsc.store_compressed(bufs[k].at[pl.ds(pos[k], 16)],
                                          packed, mask=mask)
                    pos[k] = pos[k] + jnp.sum(mask.astype(jnp.int32))
            return tuple(pos)

        pos = lax.fori_loop(0, TPW // 512, stage,
                            tuple(jnp.int32(0) for _ in range(NCH_AC)))

        dummy = jnp.full((16,), CHUNK_AC, jnp.int32)
        cvec = jnp.zeros((16,), jnp.int32)
        for k in range(NCH_AC):
            for t in range(9):
                bufs[k][pl.ds(pos[k] + t * 16, 16)] = dummy
            nb = lax.shift_right_logical(pos[k] + 127, 7)

            def flush(j, _, k=k):
                pltpu.sync_copy(
                    bufs[k].at[pl.ds(j * 128, 128)],
                    binned_out.at[wid * NCH_AC + k, pl.ds(j * 128, 128)])
                return 0
            lax.fori_loop(0, nb, flush, 0)
            cvec = jnp.where(iota == k, pos[k], cvec)
        cnt_st[pl.ds(0, 16)] = cvec
        pltpu.sync_copy(cnt_st.at[pl.ds(0, 8)], counts_out.at[wid, pl.ds(0, 8)])

    return pl.kernel(
        body, out_type=out_type, mesh=mesh, scratch_types=tuple(scratch),
        compiler_params=pltpu.CompilerParams(needs_layout_passes=False))


def _sc_segsum_binned():
    mesh = plsc.VectorSubcoreMesh(core_axis_name="c", subcore_axis_name="s",
                                  num_cores=2, num_subcores=16)
    out_type = jax.ShapeDtypeStruct((2, NCH_AC * CHUNK_AC, D), jnp.float32)
    scratch = [
        pltpu.VMEM((128,), jnp.int32),
        pltpu.VMEM((128,), jnp.int32),
        pltpu.VMEM((128,), jnp.int32),
        pltpu.VMEM((128,), jnp.int32),
        pltpu.VMEM((1, 128), jnp.int32),
        pltpu.VMEM((1, 128), jnp.int32),
        pltpu.VMEM((128, D), jnp.float32),
        pltpu.VMEM((128, D), jnp.float32),
        pltpu.VMEM((16, D), jnp.float32),
        pltpu.VMEM((16,), jnp.int32),
        pltpu.VMEM_SHARED((ACC_AC, D), jnp.float32),
        pltpu.SemaphoreType.DMA,
        pltpu.SemaphoreType.DMA,
        pltpu.SemaphoreType.DMA,
        pltpu.SemaphoreType.DMA,
        pltpu.SemaphoreType.DMA,
        pltpu.SemaphoreType.DMA,
        pltpu.SemaphoreType.DMA,
    ]

    def body(x_ref, binned_ref, counts_ref, out_ref,
             pk0, pk1, sr0, sr1, dl0, dl1, rows0, rows1, zbuf, cnt_st, acc,
             is0, is1, gs0, gs1, ss0, ss1, zs):
        pk_st = (pk0, pk1)
        src_st = (sr0, sr1)
        dstl_st = (dl0, dl1)
        rows = (rows0, rows1)
        isem = (is0, is1)
        gsem = (gs0, gs1)
        ssem = (ss0, ss1)
        c = lax.axis_index("c")
        s = lax.axis_index("s")
        w = s * 2 + c
        iota = lax.iota(jnp.int32, 16)
        zv = jnp.zeros((16,), jnp.float32)

        def fill_z(r, _):
            for l in range(8):
                zbuf[r, pl.ds(l * 16, 16)] = zv
            return 0
        lax.fori_loop(0, 16, fill_z, 0)

        wr = CHUNK_AC // 16
        for k in range(NCH_AC):
            def idx_cp(blk, b, k=k):
                return pltpu.make_async_copy(
                    binned_ref.at[w * NCH_AC + k, pl.ds(blk * 128, 128)],
                    pk_st[b], isem[b])

            def scat_cp(b):
                return pltpu.make_async_copy(
                    rows[b], acc.at[dstl_st[b].at[0]], ssem[b])

            def gath_cp(b):
                return pltpu.make_async_copy(x_ref.at[src_st[b]], rows[b],
                                             gsem[b])

            def stage_a(blk, b, nblk, wait_scat):
                idx_cp(blk, b).wait()
                if wait_scat:
                    scat_cp(b).wait()
                for l in range(8):
                    pkv = pk_st[b][pl.ds(l * 16, 16)]
                    src_st[b][pl.ds(l * 16, 16)] = lax.shift_right_logical(
                        pkv, PACK_SHIFT)
                    dstl_st[b][0, pl.ds(l * 16, 16)] = pkv & (
                        (1 << PACK_SHIFT) - 1)

                @pl.when(blk + 2 < nblk)
                def _():
                    idx_cp(blk + 2, b).start()
                gath_cp(b).start()

            def complete(b):
                gath_cp(b).wait()
                pltpu.async_copy(rows[b], acc.at[dstl_st[b].at[0]], ssem[b],
                                 add=True)

            def zcp(i):
                return pltpu.make_async_copy(
                    zbuf, acc.at[pl.ds(s * wr + i * 16, 16), :], zs)

            def zstart(i, _):
                zcp(i).start()
                return 0
            lax.fori_loop(0, wr // 16, zstart, 0)

            def zwait(i, _):
                zcp(i).wait()
                return 0
            lax.fori_loop(0, wr // 16, zwait, 0)
            plsc.subcore_barrier()

            pltpu.sync_copy(counts_ref.at[w, pl.ds(0, 8)], cnt_st.at[pl.ds(0, 8)])
            cntk = jnp.sum(jnp.where(iota == k, cnt_st[pl.ds(0, 16)], 0))
            nblk = lax.shift_right_logical(cntk + 127, 7)

            @pl.when(nblk >= 1)
            def _():
                idx_cp(0, 0).start()

            @pl.when(nblk >= 2)
            def _():
                idx_cp(1, 1).start()

            @pl.when(nblk >= 1)
            def _():
                stage_a(0, 0, nblk, False)

            @pl.when(nblk >= 2)
            def _():
                stage_a(1, 1, nblk, False)

            def pair(m, _):
                @pl.when(2 * m < nblk)
                def _():
                    complete(0)
                    stage_a(2 * m, 0, nblk, True)

                @pl.when(2 * m + 1 < nblk)
                def _():
                    complete(1)
                    stage_a(2 * m + 1, 1, nblk, True)
                return 0
            lax.fori_loop(1, (nblk + 1) // 2, pair, 0)

            @pl.when(nblk >= 1)
            def _():
                complete(0)

            @pl.when(nblk >= 2)
            def _():
                complete(1)

            @pl.when(nblk >= 1)
            def _():
                scat_cp(0).wait()

            @pl.when(nblk >= 2)
            def _():
                scat_cp(1).wait()
            plsc.subcore_barrier()

            pltpu.sync_copy(
                acc.at[pl.ds(s * wr, wr), :],
                out_ref.at[c, pl.ds(k * CHUNK_AC + s * wr, wr), :])

    return pl.kernel(
        body, out_type=out_type, mesh=mesh, scratch_types=tuple(scratch),
        compiler_params=pltpu.CompilerParams(needs_layout_passes=False))


def _sc_hist():
    mesh = plsc.VectorSubcoreMesh(core_axis_name="c", subcore_axis_name="s",
                                  num_cores=2, num_subcores=16)
    out_type = (jax.ShapeDtypeStruct((NTILES, HIST_CA), jnp.float32),
                jax.ShapeDtypeStruct((NTILES, HIST_AC), jnp.float32))
    scratch = [
        pltpu.VMEM((4, 128), jnp.int32),
        pltpu.VMEM((HIST_CA,), jnp.float32),
        pltpu.VMEM((HIST_AC,), jnp.float32),
    ]

    def body(dca_ref, dac_ref, hca_out, hac_out, dst_st, hca, hac):
        c = lax.axis_index("c")
        s = lax.axis_index("s")
        wid = s * 2 + c
        zv = jnp.zeros((16,), jnp.float32)
        ones = jnp.ones((16,), jnp.float32)

        def fill_hca(i, _):
            hca[pl.ds(i * 16, 16)] = zv
            return 0
        lax.fori_loop(0, HIST_CA // 16, fill_hca, 0)

        def fill_hac(i, _):
            hac[pl.ds(i * 16, 16)] = zv
            return 0
        lax.fori_loop(0, HIST_AC // 16, fill_hac, 0)

        for dref, hist in ((dca_ref, hca), (dac_ref, hac)):
            def blk(j, _, dref=dref, hist=hist):
                e0 = wid * RPT + j * 4
                pltpu.sync_copy(dref.at[pl.ds(e0, 4), :], dst_st)
                for i in range(4):
                    for l in range(8):
                        plsc.addupdate_scatter(
                            hist, [dst_st[i, pl.ds(l * 16, 16)]], ones)
                return 0
            lax.fori_loop(0, RPT // 4, blk, 0)
        pltpu.sync_copy(hca, hca_out.at[wid])
        pltpu.sync_copy(hac, hac_out.at[wid])

    return pl.kernel(
        body, out_type=out_type, mesh=mesh, scratch_types=tuple(scratch),
        compiler_params=pltpu.CompilerParams(needs_layout_passes=False))


def _leaky(x):
    return jnp.where(x >= 0, x, 0.1 * x)


def _tc_layer_body(p_ref, h_ref, x_ref, wl_ref, b_ref, wr_ref, o_ref):
    sums = p_ref[0] + p_ref[1]
    cnt = jnp.maximum(jnp.sum(h_ref[...], axis=0), 1.0)
    mean = sums / cnt[:, None]
    z = (jnp.dot(mean, wl_ref[...], preferred_element_type=jnp.float32)
         + b_ref[...][None, :]
         + jnp.dot(x_ref[...], wr_ref[...], preferred_element_type=jnp.float32))
    o_ref[...] = _leaky(z)


def _tc_final_body(p_ref, h_ref, x_ref, wl_ref, b_ref, wr_ref, wlin_ref,
                   blin_ref, o_ref):
    sums = p_ref[0] + p_ref[1]
    cnt = jnp.maximum(jnp.sum(h_ref[...], axis=0), 1.0)
    mean = sums / cnt[:, None]
    z = (jnp.dot(mean, wl_ref[...], preferred_element_type=jnp.float32)
         + b_ref[...][None, :]
         + jnp.dot(x_ref[...], wr_ref[...], preferred_element_type=jnp.float32))
    o_ref[...] = (jnp.dot(_leaky(z), wlin_ref[...],
                          preferred_element_type=jnp.float32)
                  + blin_ref[...][None, :])


def _tc_layer(n_out):
    grid = ((n_out + BLKR - 1) // BLKR,)
    return pl.pallas_call(
        _tc_layer_body,
        grid=grid,
        in_specs=[
            pl.BlockSpec((2, BLKR, D), lambda i: (0, i, 0)),
            pl.BlockSpec((NTILES, BLKR), lambda i: (0, i)),
            pl.BlockSpec((BLKR, D), lambda i: (i, 0)),
            pl.BlockSpec((D, D), lambda i: (0, 0)),
            pl.BlockSpec((D,), lambda i: (0,)),
            pl.BlockSpec((D, D), lambda i: (0, 0)),
        ],
        out_specs=pl.BlockSpec((BLKR, D), lambda i: (i, 0)),
        out_shape=jax.ShapeDtypeStruct((n_out, D), jnp.float32),
    )


def _tc_final(n_out):
    grid = ((n_out + BLKR - 1) // BLKR,)
    return pl.pallas_call(
        _tc_final_body,
        grid=grid,
        in_specs=[
            pl.BlockSpec((2, BLKR, D), lambda i: (0, i, 0)),
            pl.BlockSpec((NTILES, BLKR), lambda i: (0, i)),
            pl.BlockSpec((BLKR, D), lambda i: (i, 0)),
            pl.BlockSpec((D, D), lambda i: (0, 0)),
            pl.BlockSpec((D,), lambda i: (0,)),
            pl.BlockSpec((D, D), lambda i: (0, 0)),
            pl.BlockSpec((D, 1), lambda i: (0, 0)),
            pl.BlockSpec((1,), lambda i: (0,)),
        ],
        out_specs=pl.BlockSpec((BLKR, 1), lambda i: (i, 0)),
        out_shape=jax.ShapeDtypeStruct((n_out, 1), jnp.float32),
    )


_sc_h = _sc_hist()
_sc_ca = _sc_segsum(1, ACC_CA, ACC_CA, False)
_sc_b = _sc_bin()
_sc_ac = _sc_segsum_binned()
_t1 = _tc_layer(N_A)
_t2 = _tc_layer(N_C)
_t3 = _tc_final(N_C)


def kernel(x_clients, x_aggregator, edge_ca, edge_ac,
           W1_ca_l, b1_ca, W1_ca_r, W1_ac_l, b1_ac, W1_ac_r,
           W2_ca_l, b2_ca, W2_ca_r, W2_ac_l, b2_ac, W2_ac_r,
           W_lin, b_lin):
    pad = EPAD - E
    ec = edge_ca.astype(jnp.int32)
    ea = edge_ac.astype(jnp.int32)
    src_ca = jnp.pad(ec[0], (0, pad))
    dst_ca = jnp.pad(ec[1], (0, pad),
                     constant_values=DUMP_CA).reshape(EROWS, 128)
    src_ac = jnp.pad(ea[0], (0, pad))
    dst_ac1 = jnp.pad(ea[1], (0, pad), constant_values=DUMP_AC)
    dst_ac = dst_ac1.reshape(EROWS, 128)

    hist_ca, hist_ac = _sc_h(dst_ca, dst_ac)
    binned, bcounts = _sc_b(src_ac, dst_ac1)
    sums_ca = _sc_ca(x_clients, src_ca, dst_ca)
    sums_ac1 = _sc_ac(x_aggregator, binned, bcounts)
    a = _t1(sums_ca, hist_ca, x_aggregator, W1_ca_l, b1_ca, W1_ca_r)
    c = _t2(sums_ac1, hist_ac, x_clients, W1_ac_l, b1_ac, W1_ac_r)
    sums_ac2 = _sc_ac(a, binned, bcounts)
    out = _t3(sums_ac2, hist_ac, c, W2_ac_l, b2_ac, W2_ac_r, W_lin, b_lin)
    return jnp.squeeze(out, axis=1)
